# Initial kernel scaffold; baseline (speedup 1.0000x reference)
import jax
import jax.numpy as jnp
from jax.experimental import pallas as pl


def kernel(data, v_bins):
    raise NotImplementedError("write your pallas kernel here")


# speedup vs baseline: 118.1818x; 118.1818x over previous
import jax
import jax.numpy as jnp
from jax.experimental import pallas as pl


def _onehot_body(x_ref, lo_ref, hi_ref, o_ref):
    x = x_ref[...]
    lo = lo_ref[...]
    hi = hi_ref[...]
    hit = jnp.logical_and(x > lo, x <= hi)
    o_ref[...] = hit.astype(jnp.float32)


def kernel(data, v_bins):
    n_bins = v_bins.shape[0] - 1
    n = data.size
    x = data.reshape(n, 1)

    mid = v_bins[1:n_bins]
    lo = jnp.concatenate([jnp.full((1,), -jnp.inf, v_bins.dtype), mid]).reshape(1, n_bins)
    hi = jnp.concatenate([mid, jnp.full((1,), jnp.inf, v_bins.dtype)]).reshape(1, n_bins)

    block = 16384
    grid = n // block
    out = pl.pallas_call(
        _onehot_body,
        grid=(grid,),
        in_specs=[
            pl.BlockSpec((block, 1), lambda i: (i, 0)),
            pl.BlockSpec((1, n_bins), lambda i: (0, 0)),
            pl.BlockSpec((1, n_bins), lambda i: (0, 0)),
        ],
        out_specs=pl.BlockSpec((block, n_bins), lambda i: (i, 0)),
        out_shape=jax.ShapeDtypeStruct((n, n_bins), jnp.float32),
    )(x, lo, hi)
    return out.reshape(data.shape + (n_bins,))
